# Initial kernel scaffold; baseline (speedup 1.0000x reference)
import jax
import jax.numpy as jnp
from jax.experimental import pallas as pl


def kernel(x, emb):
    raise NotImplementedError("write your pallas kernel here")


# speedup vs baseline: 2.9702x; 2.9702x over previous
import functools

import numpy as np
import jax
import jax.numpy as jnp
from jax import lax
from jax.experimental import pallas as pl
from jax.experimental.pallas import tpu as pltpu
from jax.experimental.pallas import tpu_sc as plsc

D_MODEL = 64
SEQ = 200
SCALE = 8.0

_info = plsc.get_sparse_core_info()
_NC, _NS, _L = _info.num_cores, _info.num_subcores, _info.num_lanes
_NW = _NC * _NS

CHUNK = 800
REPS = CHUNK // SEQ
D_VREGS = D_MODEL // 16


def _pe_slice():
    pos = np.expand_dims(np.arange(0, SEQ), axis=1)
    div_term = np.array(
        [[1 / np.power(10000, 2 * (i // 2) / D_MODEL) for i in range(D_MODEL)]]
    )
    p = pos * div_term
    pe = np.zeros((SEQ, D_MODEL), dtype=np.float32)
    pe[:, 0::2] = np.sin(p[:, 0::2])
    pe[:, 1::2] = np.cos(p[:, 0::2])
    return pe.reshape(-1)


_PE = _pe_slice()


@functools.partial(jax.jit, static_argnames=("n_rows",))
def _embed(xf, emb, pe, n_rows):
    rows_per_w = n_rows // _NW
    n_chunks = rows_per_w // CHUNK
    mesh = plsc.VectorSubcoreMesh(core_axis_name="c", subcore_axis_name="s")

    @functools.partial(
        pl.kernel,
        mesh=mesh,
        out_type=jax.ShapeDtypeStruct((n_rows, D_MODEL), jnp.float32),
        scratch_types=[
            pltpu.VMEM((CHUNK,), jnp.int32),
            pltpu.VMEM((CHUNK, D_MODEL), jnp.float32),
            pltpu.VMEM((SEQ * D_MODEL,), jnp.float32),
            pltpu.SemaphoreType.DMA,
        ],
        compiler_params=pltpu.CompilerParams(use_tc_tiling_on_sc=False),
    )
    def k(x_hbm, emb_hbm, pe_hbm, out_hbm, idx_v, rows_v, pe_v, sem):
        wid = lax.axis_index("s") * _NC + lax.axis_index("c")
        base = wid * rows_per_w
        pltpu.sync_copy(pe_hbm, pe_v)

        def chunk_body(c, carry):
            cb = base + c * CHUNK
            pltpu.sync_copy(x_hbm.at[pl.ds(cb, CHUNK)], idx_v)
            pltpu.async_copy(emb_hbm.at[idx_v], rows_v, sem).wait()

            def row_body(r, carry2):
                for d in range(D_VREGS):
                    pe_vec = pe_v[pl.ds((r * D_VREGS + d) * 16, 16)]
                    for rep in range(REPS):
                        row = rep * SEQ + r
                        sl = pl.ds(d * 16, 16)
                        rows_v[row, sl] = rows_v[row, sl] * SCALE + pe_vec
                return carry2

            lax.fori_loop(0, SEQ, row_body, 0)
            pltpu.sync_copy(rows_v, out_hbm.at[pl.ds(cb, CHUNK)])
            return carry

        lax.fori_loop(0, n_chunks, chunk_body, 0)

    return k(xf, emb, pe)


def kernel(x, emb):
    b, s = x.shape
    xf = x.reshape(-1)
    out = _embed(xf, emb, _PE, b * s)
    return out.reshape(b, s, D_MODEL)


# speedup vs baseline: 3.2583x; 1.0970x over previous
import functools

import numpy as np
import jax
import jax.numpy as jnp
from jax import lax
from jax.experimental import pallas as pl
from jax.experimental.pallas import tpu as pltpu
from jax.experimental.pallas import tpu_sc as plsc

D_MODEL = 64
SEQ = 200
SCALE = 8.0

_info = plsc.get_sparse_core_info()
_NC, _NS, _L = _info.num_cores, _info.num_subcores, _info.num_lanes
_NW = _NC * _NS

CHUNK = 800
REPS = CHUNK // SEQ
D_VREGS = D_MODEL // 16


def _pe_slice():
    pos = np.expand_dims(np.arange(0, SEQ), axis=1)
    div_term = np.array(
        [[1 / np.power(10000, 2 * (i // 2) / D_MODEL) for i in range(D_MODEL)]]
    )
    p = pos * div_term
    pe = np.zeros((SEQ, D_MODEL), dtype=np.float32)
    pe[:, 0::2] = np.sin(p[:, 0::2])
    pe[:, 1::2] = np.cos(p[:, 0::2])
    return pe.reshape(-1)


_PE = _pe_slice()


@functools.partial(jax.jit, static_argnames=("n_rows",))
def _embed(xf, emb, pe, n_rows):
    rows_per_w = n_rows // _NW
    n_chunks = rows_per_w // CHUNK
    mesh = plsc.VectorSubcoreMesh(core_axis_name="c", subcore_axis_name="s")

    @functools.partial(
        pl.kernel,
        mesh=mesh,
        out_type=jax.ShapeDtypeStruct((n_rows, D_MODEL), jnp.float32),
        scratch_types=[
            pltpu.VMEM((rows_per_w,), jnp.int32),
            pltpu.VMEM((CHUNK, D_MODEL), jnp.float32),
            pltpu.VMEM((CHUNK, D_MODEL), jnp.float32),
            pltpu.VMEM((SEQ * D_MODEL,), jnp.float32),
            pltpu.SemaphoreType.DMA,
            pltpu.SemaphoreType.DMA,
        ],
        compiler_params=pltpu.CompilerParams(use_tc_tiling_on_sc=False),
    )
    def k(x_hbm, emb_hbm, pe_hbm, out_hbm, idx_v, rows0, rows1, pe_v, gsem, ssem):
        wid = lax.axis_index("s") * _NC + lax.axis_index("c")
        base = wid * rows_per_w
        pltpu.sync_copy(x_hbm.at[pl.ds(base, rows_per_w)], idx_v)
        pltpu.sync_copy(pe_hbm, pe_v)
        bufs = (rows0, rows1)

        def gather(c):
            return pltpu.async_copy(
                emb_hbm.at[idx_v.at[pl.ds(c * CHUNK, CHUNK)]], bufs[c % 2], gsem
            )

        def store(c):
            return pltpu.async_copy(
                bufs[c % 2], out_hbm.at[pl.ds(base + c * CHUNK, CHUNK)], ssem
            )

        def compute(buf):
            def row_body(r, carry):
                for d in range(D_VREGS):
                    pe_vec = pe_v[pl.ds((r * D_VREGS + d) * 16, 16)]
                    for rep in range(REPS):
                        row = rep * SEQ + r
                        sl = pl.ds(d * 16, 16)
                        buf[row, sl] = buf[row, sl] * SCALE + pe_vec
                return carry

            lax.fori_loop(0, SEQ, row_body, 0)

        gathers = {0: gather(0)}
        stores = {}
        for c in range(n_chunks):
            gathers[c].wait()
            if c + 1 < n_chunks:
                if c >= 1:
                    stores[c - 1].wait()
                gathers[c + 1] = gather(c + 1)
            compute(bufs[c % 2])
            stores[c] = store(c)
        stores[n_chunks - 2].wait()
        stores[n_chunks - 1].wait()

    return k(xf, emb, pe)


def kernel(x, emb):
    b, s = x.shape
    xf = x.reshape(-1)
    out = _embed(xf, emb, _PE, b * s)
    return out.reshape(b, s, D_MODEL)
